# baseline (speedup 1.0000x reference)
import functools

import jax
import jax.numpy as jnp
from jax import lax
from jax.experimental import pallas as pl
from jax.experimental.pallas import tpu as pltpu
from jax.experimental.pallas import tpu_sc as plsc

V = 10000
C = 128
FOUT = 128
E = 320000
NC = 2
NS = 16
NW = NC * NS
EPT = E // NW
CHUNK = 128
NPC = -(-EPT // CHUNK)
GROUPS = 4
NPAIR = CHUNK // 2

_mesh = plsc.VectorSubcoreMesh(core_axis_name="c", subcore_axis_name="s",
                               num_cores=NC, num_subcores=NS)


@functools.partial(
    pl.kernel,
    out_type=jax.ShapeDtypeStruct((NC * V, C), jnp.bfloat16),
    mesh=_mesh,
    compiler_params=pltpu.CompilerParams(use_tc_tiling_on_sc=False),
    scratch_types=dict(
        accum=pltpu.VMEM_SHARED((V, C), jnp.bfloat16),
        col_v=pltpu.VMEM((NPC, CHUNK), jnp.int32),
        row_v=pltpu.VMEM((NPC, CHUNK), jnp.int32),
        rows_a=pltpu.VMEM((CHUNK, C), jnp.bfloat16),
        rows_b=pltpu.VMEM((CHUNK, C), jnp.bfloat16),
        wx_a=pltpu.VMEM((NPAIR, 2, 16), jnp.bfloat16),
        wx_b=pltpu.VMEM((NPAIR, 2, 16), jnp.bfloat16),
        ga=pltpu.SemaphoreType.DMA,
        gb=pltpu.SemaphoreType.DMA,
        sa=pltpu.SemaphoreType.DMA,
        sb=pltpu.SemaphoreType.DMA,
        wa=pltpu.SemaphoreType.DMA,
        wb=pltpu.SemaphoreType.DMA,
    ),
)
def _spmv_sc(x_hbm, row_hbm, col_hbm, w_hbm, out_hbm,
             accum, col_v, row_v, rows_a, rows_b, wx_a, wx_b,
             ga, gb, sa, sb, wa, wb):
    cid = lax.axis_index("c")
    sid = lax.axis_index("s")
    wid = sid * NC + cid

    rows = (rows_a, rows_b)
    wx = (wx_a, wx_b)
    gsem = (ga, gb)
    ssem = (sa, sb)
    wsem = (wa, wb)

    pltpu.sync_copy(col_hbm.at[wid], col_v)
    pltpu.sync_copy(row_hbm.at[wid], row_v)

    zero32 = jnp.zeros((32,), jnp.bfloat16)

    @pl.loop(0, CHUNK)
    def _zb(j):
        for c in range(GROUPS):
            rows_a[j, pl.ds(c * 32, 32)] = zero32

    @pl.loop(0, 4)
    def _za(k):
        pltpu.sync_copy(rows_a, accum.at[pl.ds(sid * 624 + k * CHUNK, CHUNK)])

    pltpu.sync_copy(rows_a.at[pl.ds(0, 112)],
                    accum.at[pl.ds(sid * 624 + 512, 112)])

    @pl.when(sid == NS - 1)
    def _ztail():
        pltpu.sync_copy(rows_a.at[pl.ds(0, 16)], accum.at[pl.ds(9984, 16)])

    plsc.subcore_barrier()

    def g_start(it, b):
        pltpu.async_copy(x_hbm.at[col_v.at[it]], rows[b], gsem[b])
        pltpu.async_copy(w_hbm.at[wid, it], wx[b], wsem[b])

    def g_wait(b):
        pltpu.make_async_copy(x_hbm.at[col_v.at[0]], rows[b], gsem[b]).wait()
        pltpu.make_async_copy(w_hbm.at[0, 0], wx[b], wsem[b]).wait()

    def s_start(it, b):
        pltpu.async_copy(rows[b], accum.at[row_v.at[it]], ssem[b], add=True)

    def s_wait(b):
        pltpu.make_async_copy(rows[b], accum.at[row_v.at[0]], ssem[b]).wait()

    def scale(it, b):
        rbuf = rows[b]
        wbuf = wx[b]

        @pl.loop(0, NPAIR)
        def _scale(jp):
            jb = pl.multiple_of(2 * jp, 2)
            wbf = wbuf[jp]
            for c in range(GROUPS * 2):
                sl = (pl.ds(jb, 2), pl.ds(c * 16, 16))
                rbuf[sl] = rbuf[sl] * wbf

    def handle(it, b):
        o = 1 - b
        s_wait(o)

        @pl.when(it + 1 < NPC)
        def _pref():
            g_start(it + 1, o)

        g_wait(b)
        scale(it, b)
        s_start(it, b)

    g_start(0, 0)
    g_wait(0)
    g_start(1, 1)
    scale(0, 0)
    s_start(0, 0)

    @pl.loop(0, (NPC - 1) // 2)
    def _pipe(i):
        handle(1 + 2 * i, 1)
        handle(2 + 2 * i, 0)

    if (NPC - 1) % 2 == 1:
        handle(NPC - 1, 1)
    s_wait((NPC - 1) % 2)

    plsc.subcore_barrier()

    pltpu.sync_copy(accum.at[pl.ds(sid * 624, 624)],
                    out_hbm.at[pl.ds(cid * V + sid * 624, 624)])

    @pl.when(sid == NS - 1)
    def _tail():
        pltpu.sync_copy(accum.at[pl.ds(9984, 16)],
                        out_hbm.at[pl.ds(cid * V + 9984, 16)])


_RB = 1000


def _combine_body(a_ref, b_ref, o_ref, obf_ref):
    s = a_ref[...].astype(jnp.float32) + b_ref[...].astype(jnp.float32)
    o_ref[...] = s
    obf_ref[...] = s.astype(jnp.bfloat16)


def _combine(p):
    return pl.pallas_call(
        _combine_body,
        grid=(V // _RB,),
        in_specs=[
            pl.BlockSpec((_RB, C), lambda i: (i, 0)),
            pl.BlockSpec((_RB, C), lambda i: (i + V // _RB, 0)),
        ],
        out_specs=[
            pl.BlockSpec((_RB, C), lambda i: (i, 0)),
            pl.BlockSpec((_RB, C), lambda i: (i, 0)),
        ],
        out_shape=[
            jax.ShapeDtypeStruct((V, C), jnp.float32),
            jax.ShapeDtypeStruct((V, C), jnp.bfloat16),
        ],
    )(p, p)


def _mix_body(x0_ref, x1_ref, p2a_ref, p2b_ref, w_ref, o_ref):
    x0b = x0_ref[...]
    x1b = x1_ref[...]
    x2b = (2.0 * (p2a_ref[...].astype(jnp.float32)
                  + p2b_ref[...].astype(jnp.float32)) - x0b)
    acc = jnp.dot(x0b, w_ref[0], preferred_element_type=jnp.float32)
    acc += jnp.dot(x1b, w_ref[1], preferred_element_type=jnp.float32)
    acc += jnp.dot(x2b, w_ref[2], preferred_element_type=jnp.float32)
    o_ref[...] = acc


def _mix(x0, x1, p2, weight):
    return pl.pallas_call(
        _mix_body,
        grid=(V // _RB,),
        in_specs=[
            pl.BlockSpec((_RB, C), lambda i: (i, 0)),
            pl.BlockSpec((_RB, C), lambda i: (i, 0)),
            pl.BlockSpec((_RB, C), lambda i: (i, 0)),
            pl.BlockSpec((_RB, C), lambda i: (i + V // _RB, 0)),
            pl.BlockSpec((3, C, FOUT), lambda i: (0, 0, 0)),
        ],
        out_specs=pl.BlockSpec((_RB, FOUT), lambda i: (i, 0)),
        out_shape=jax.ShapeDtypeStruct((V, FOUT), jnp.float32),
    )(x0, x1, p2, p2, weight)


def _pad_edges(a, fill):
    per = a.reshape(NW, EPT)
    pad = jnp.full((NW, NPC * CHUNK - EPT), fill, a.dtype)
    return jnp.concatenate([per, pad], axis=1).reshape(NW, NPC, CHUNK)


def kernel(inputs, edge_index, edge_weight, weight):
    B, Fin, V_, X, Y, Z = inputs.shape
    K, _, Fout = weight.shape
    x0 = inputs.reshape(Fin, V_).T
    x0bf = x0.astype(jnp.bfloat16)
    row = _pad_edges(edge_index[0], 0)
    col = _pad_edges(edge_index[1], 0)
    w3 = _pad_edges(edge_weight, 0.0)
    wexp = jnp.broadcast_to(w3.astype(jnp.bfloat16)[..., None],
                            (NW, NPC, CHUNK, 16)).reshape(NW, NPC, NPAIR, 2, 16)
    p1 = _spmv_sc(x0bf, row, col, wexp)
    x1, x1bf = _combine(p1)
    p2 = _spmv_sc(x1bf, row, col, wexp)
    y = _mix(x0, x1, p2, weight)
    return y.T.reshape(B, Fout, V_, X, Y, Z)


# speedup vs baseline: 1.7748x; 1.7748x over previous
import functools

import jax
import jax.numpy as jnp
from jax import lax
from jax.experimental import pallas as pl
from jax.experimental.pallas import tpu as pltpu
from jax.experimental.pallas import tpu_sc as plsc

V = 10000
C = 128
FOUT = 128
E = 320000
NC = 2
NS = 16
NW = NC * NS
EPT = E // NW
CHUNK = 128
NPC = -(-EPT // CHUNK)
GROUPS = 4
NBUF = 4

_mesh = plsc.VectorSubcoreMesh(core_axis_name="c", subcore_axis_name="s",
                               num_cores=NC, num_subcores=NS)

_scratch = dict(
    accum=pltpu.VMEM_SHARED((V, C), jnp.bfloat16),
    col_v=pltpu.VMEM((NPC, CHUNK), jnp.int32),
    row_v=pltpu.VMEM((NPC, CHUNK), jnp.int32),
)
for _b in range(NBUF):
    _scratch[f"rows{_b}"] = pltpu.VMEM((CHUNK, C), jnp.bfloat16)
    _scratch[f"wx{_b}"] = pltpu.VMEM((CHUNK, 32), jnp.bfloat16)
for _b in range(NBUF):
    _scratch[f"g{_b}"] = pltpu.SemaphoreType.DMA
    _scratch[f"s{_b}"] = pltpu.SemaphoreType.DMA
    _scratch[f"w{_b}"] = pltpu.SemaphoreType.DMA


@functools.partial(
    pl.kernel,
    out_type=jax.ShapeDtypeStruct((NC * V, C), jnp.bfloat16),
    mesh=_mesh,
    compiler_params=pltpu.CompilerParams(use_tc_tiling_on_sc=False),
    scratch_types=_scratch,
)
def _spmv_sc(x_hbm, row_hbm, col_hbm, w_hbm, out_hbm,
             accum, col_v, row_v,
             rows0, wx0, rows1, wx1, rows2, wx2, rows3, wx3,
             g0, s0, w0, g1, s1, w1, g2, s2, w2, g3, s3, w3):
    cid = lax.axis_index("c")
    sid = lax.axis_index("s")
    wid = sid * NC + cid

    rows = (rows0, rows1, rows2, rows3)
    wx = (wx0, wx1, wx2, wx3)
    gsem = (g0, g1, g2, g3)
    ssem = (s0, s1, s2, s3)
    wsem = (w0, w1, w2, w3)

    pltpu.sync_copy(col_hbm.at[wid], col_v)
    pltpu.sync_copy(row_hbm.at[wid], row_v)

    zero32 = jnp.zeros((32,), jnp.bfloat16)

    @pl.loop(0, CHUNK)
    def _zb(j):
        for c in range(GROUPS):
            rows[0][j, pl.ds(c * 32, 32)] = zero32

    @pl.loop(0, 4)
    def _za(k):
        pltpu.sync_copy(rows[0], accum.at[pl.ds(sid * 624 + k * CHUNK, CHUNK)])

    pltpu.sync_copy(rows[0].at[pl.ds(0, 112)],
                    accum.at[pl.ds(sid * 624 + 512, 112)])

    @pl.when(sid == NS - 1)
    def _ztail():
        pltpu.sync_copy(rows[0].at[pl.ds(0, 16)], accum.at[pl.ds(9984, 16)])

    plsc.subcore_barrier()

    def g_start(it, b):
        pltpu.async_copy(x_hbm.at[col_v.at[it]], rows[b], gsem[b])
        pltpu.async_copy(w_hbm.at[wid, it], wx[b], wsem[b])

    def g_wait(b):
        pltpu.make_async_copy(x_hbm.at[col_v.at[0]], rows[b], gsem[b]).wait()
        pltpu.make_async_copy(w_hbm.at[0, 0], wx[b], wsem[b]).wait()

    def s_start(it, b):
        pltpu.async_copy(rows[b], accum.at[row_v.at[it]], ssem[b], add=True)

    def s_wait(b):
        pltpu.make_async_copy(rows[b], accum.at[row_v.at[0]], ssem[b]).wait()

    def scale(it, b):
        rbuf = rows[b]
        wbuf = wx[b]

        @pl.loop(0, CHUNK)
        def _scale(j):
            wbf = wbuf[j]
            for c in range(GROUPS):
                sl = pl.ds(c * 32, 32)
                rbuf[j, sl] = rbuf[j, sl] * wbf

    def steady(it, b):
        g_wait(b)
        scale(it, b)
        s_start(it, b)
        nb = (b + NBUF - 1) % NBUF
        s_wait(nb)
        g_start(it + NBUF - 1, nb)

    def tail(it, b):
        g_wait(b)
        scale(it, b)
        s_start(it, b)

    for k in range(NBUF - 1):
        g_start(k, k)
    g_wait(0)
    scale(0, 0)
    s_start(0, 0)
    g_start(NBUF - 1, NBUF - 1)

    NMAIN = NPC - NBUF
    NLOOP = (NMAIN // NBUF) * NBUF

    @pl.loop(0, NMAIN // NBUF)
    def _pipe(i):
        for k in range(NBUF):
            it = 1 + NBUF * i + k
            steady(it, (1 + k) % NBUF)

    for it in range(1 + NLOOP, NPC - NBUF + 1):
        steady(it, it % NBUF)
    for it in range(NPC - NBUF + 1, NPC):
        tail(it, it % NBUF)
    for it in range(NPC - NBUF, NPC):
        s_wait(it % NBUF)

    plsc.subcore_barrier()

    pltpu.sync_copy(accum.at[pl.ds(sid * 624, 624)],
                    out_hbm.at[pl.ds(cid * V + sid * 624, 624)])

    @pl.when(sid == NS - 1)
    def _tail():
        pltpu.sync_copy(accum.at[pl.ds(9984, 16)],
                        out_hbm.at[pl.ds(cid * V + 9984, 16)])


_RB = 1000


def _combine_body(a_ref, b_ref, o_ref, obf_ref):
    s = a_ref[...].astype(jnp.float32) + b_ref[...].astype(jnp.float32)
    o_ref[...] = s
    obf_ref[...] = s.astype(jnp.bfloat16)


def _combine(p):
    return pl.pallas_call(
        _combine_body,
        grid=(V // _RB,),
        in_specs=[
            pl.BlockSpec((_RB, C), lambda i: (i, 0)),
            pl.BlockSpec((_RB, C), lambda i: (i + V // _RB, 0)),
        ],
        out_specs=[
            pl.BlockSpec((_RB, C), lambda i: (i, 0)),
            pl.BlockSpec((_RB, C), lambda i: (i, 0)),
        ],
        out_shape=[
            jax.ShapeDtypeStruct((V, C), jnp.float32),
            jax.ShapeDtypeStruct((V, C), jnp.bfloat16),
        ],
    )(p, p)


def _mix_body(x0_ref, x1_ref, p2a_ref, p2b_ref, w_ref, o_ref):
    x0b = x0_ref[...]
    x1b = x1_ref[...]
    x2b = (2.0 * (p2a_ref[...].astype(jnp.float32)
                  + p2b_ref[...].astype(jnp.float32)) - x0b)
    acc = jnp.dot(x0b, w_ref[0], preferred_element_type=jnp.float32)
    acc += jnp.dot(x1b, w_ref[1], preferred_element_type=jnp.float32)
    acc += jnp.dot(x2b, w_ref[2], preferred_element_type=jnp.float32)
    o_ref[...] = acc


def _mix(x0, x1, p2, weight):
    return pl.pallas_call(
        _mix_body,
        grid=(V // _RB,),
        in_specs=[
            pl.BlockSpec((_RB, C), lambda i: (i, 0)),
            pl.BlockSpec((_RB, C), lambda i: (i, 0)),
            pl.BlockSpec((_RB, C), lambda i: (i, 0)),
            pl.BlockSpec((_RB, C), lambda i: (i + V // _RB, 0)),
            pl.BlockSpec((3, C, FOUT), lambda i: (0, 0, 0)),
        ],
        out_specs=pl.BlockSpec((_RB, FOUT), lambda i: (i, 0)),
        out_shape=jax.ShapeDtypeStruct((V, FOUT), jnp.float32),
    )(x0, x1, p2, p2, weight)


def _pad_edges(a, fill):
    per = a.reshape(NW, EPT)
    pad = jnp.full((NW, NPC * CHUNK - EPT), fill, a.dtype)
    return jnp.concatenate([per, pad], axis=1).reshape(NW, NPC, CHUNK)


def kernel(inputs, edge_index, edge_weight, weight):
    B, Fin, V_, X, Y, Z = inputs.shape
    K, _, Fout = weight.shape
    x0 = inputs.reshape(Fin, V_).T
    x0bf = x0.astype(jnp.bfloat16)
    row = _pad_edges(edge_index[0], 0)
    col = _pad_edges(edge_index[1], 0)
    w3 = _pad_edges(edge_weight, 0.0)
    wexp = jnp.broadcast_to(w3.astype(jnp.bfloat16)[..., None],
                            (NW, NPC, CHUNK, 32))
    p1 = _spmv_sc(x0bf, row, col, wexp)
    x1, x1bf = _combine(p1)
    p2 = _spmv_sc(x1bf, row, col, wexp)
    y = _mix(x0, x1, p2, weight)
    return y.T.reshape(B, Fout, V_, X, Y, Z)


# speedup vs baseline: 1.8400x; 1.0368x over previous
import functools

import jax
import jax.numpy as jnp
from jax import lax
from jax.experimental import pallas as pl
from jax.experimental.pallas import tpu as pltpu
from jax.experimental.pallas import tpu_sc as plsc

V = 10000
C = 128
FOUT = 128
E = 320000
NC = 2
NS = 16
NW = NC * NS
EPT = E // NW
CHUNK = 128
NPC = -(-EPT // CHUNK)
BLK = (NPC + 1) // 2
LANES = 8

_mesh = plsc.VectorSubcoreMesh(core_axis_name="c", subcore_axis_name="s",
                               num_cores=NC, num_subcores=NS)


@functools.partial(
    pl.kernel,
    out_type=jax.ShapeDtypeStruct((NC * V, C), jnp.float32),
    mesh=_mesh,
    scratch_types=dict(
        accum=pltpu.VMEM_SHARED((V, C), jnp.float32),
        col_v=pltpu.VMEM((BLK, CHUNK), jnp.int32),
        row_v=pltpu.VMEM((BLK, CHUNK), jnp.int32),
        w_v=pltpu.VMEM((BLK, CHUNK), jnp.float32),
        rows_a=pltpu.VMEM((CHUNK, C), jnp.float32),
        rows_b=pltpu.VMEM((CHUNK, C), jnp.float32),
        ga=pltpu.SemaphoreType.DMA,
        gb=pltpu.SemaphoreType.DMA,
        sa=pltpu.SemaphoreType.DMA,
        sb=pltpu.SemaphoreType.DMA,
    ),
)
def _spmv_sc(x_hbm, row_hbm, col_hbm, w_hbm, out_hbm,
             accum, col_v, row_v, w_v, rows_a, rows_b, ga, gb, sa, sb):
    cid = lax.axis_index("c")
    sid = lax.axis_index("s")
    wid = sid * NC + cid

    rows = (rows_a, rows_b)
    gsem = (ga, gb)
    ssem = (sa, sb)

    zero16 = jnp.zeros((16,), jnp.float32)

    @pl.loop(0, CHUNK)
    def _zb(j):
        for c in range(LANES):
            rows_a[j, pl.ds(c * 16, 16)] = zero16

    @pl.loop(0, 4)
    def _za(k):
        pltpu.sync_copy(rows_a, accum.at[pl.ds(sid * 625 + k * CHUNK, CHUNK)])

    pltpu.sync_copy(rows_a.at[pl.ds(0, 113)],
                    accum.at[pl.ds(sid * 625 + 512, 113)])

    plsc.subcore_barrier()

    def g_start(it, b):
        pltpu.async_copy(x_hbm.at[col_v.at[it]], rows[b], gsem[b])

    def g_wait(b):
        pltpu.make_async_copy(x_hbm.at[col_v.at[0]], rows[b], gsem[b]).wait()

    def s_start(it, b):
        pltpu.async_copy(rows[b], accum.at[row_v.at[it]], ssem[b], add=True)

    def s_wait(b):
        pltpu.make_async_copy(rows[b], accum.at[row_v.at[0]], ssem[b]).wait()

    def scale(it, b):
        rbuf = rows[b]

        @pl.loop(0, CHUNK // 16)
        def _scale(g):
            wvec = w_v[it, pl.ds(g * 16, 16)]
            for lane in range(16):
                wv = lax.gather(
                    wvec, jnp.full((16, 1), lane, jnp.int32),
                    lax.GatherDimensionNumbers(offset_dims=(),
                                               collapsed_slice_dims=(0,),
                                               start_index_map=(0,)),
                    slice_sizes=(1,),
                    mode=lax.GatherScatterMode.PROMISE_IN_BOUNDS)
                j = g * 16 + lane
                for c in range(LANES):
                    sl = pl.ds(c * 16, 16)
                    rbuf[j, sl] = rbuf[j, sl] * wv

    def run_block(n):
        def handle(it, b):
            o = 1 - b
            s_wait(o)

            @pl.when(it + 1 < n)
            def _pref():
                g_start(it + 1, o)

            g_wait(b)
            scale(it, b)
            s_start(it, b)

        g_start(0, 0)
        g_wait(0)
        g_start(1, 1)
        scale(0, 0)
        s_start(0, 0)

        @pl.loop(0, (n - 1) // 2)
        def _pipe(i):
            handle(1 + 2 * i, 1)
            handle(2 + 2 * i, 0)

        if (n - 1) % 2 == 1:
            handle(n - 1, 1)
        s_wait((n - 1) % 2)

    for c0, n in ((0, BLK), (BLK, NPC - BLK)):
        pltpu.sync_copy(col_hbm.at[wid, pl.ds(c0, n)], col_v.at[pl.ds(0, n)])
        pltpu.sync_copy(row_hbm.at[wid, pl.ds(c0, n)], row_v.at[pl.ds(0, n)])
        pltpu.sync_copy(w_hbm.at[wid, pl.ds(c0, n)], w_v.at[pl.ds(0, n)])
        run_block(n)

    plsc.subcore_barrier()

    pltpu.sync_copy(accum.at[pl.ds(sid * 624, 624)],
                    out_hbm.at[pl.ds(cid * V + sid * 624, 624)])

    @pl.when(sid == NS - 1)
    def _tail():
        pltpu.sync_copy(accum.at[pl.ds(15 * 624, 640)],
                        out_hbm.at[pl.ds(cid * V + 15 * 624, 640)])


_RB = 1000


def _combine_body(a_ref, b_ref, o_ref):
    o_ref[...] = a_ref[...] + b_ref[...]


def _combine(p):
    return pl.pallas_call(
        _combine_body,
        grid=(V // _RB,),
        in_specs=[
            pl.BlockSpec((_RB, C), lambda i: (i, 0)),
            pl.BlockSpec((_RB, C), lambda i: (i + V // _RB, 0)),
        ],
        out_specs=pl.BlockSpec((_RB, C), lambda i: (i, 0)),
        out_shape=jax.ShapeDtypeStruct((V, C), jnp.float32),
    )(p, p)


def _mix_body(x0_ref, x1_ref, p2a_ref, p2b_ref, w_ref, o_ref):
    x0b = x0_ref[...]
    x1b = x1_ref[...]
    x2b = 2.0 * (p2a_ref[...] + p2b_ref[...]) - x0b
    acc = jnp.dot(x0b, w_ref[0], preferred_element_type=jnp.float32)
    acc += jnp.dot(x1b, w_ref[1], preferred_element_type=jnp.float32)
    acc += jnp.dot(x2b, w_ref[2], preferred_element_type=jnp.float32)
    o_ref[...] = acc


def _mix(x0, x1, p2, weight):
    return pl.pallas_call(
        _mix_body,
        grid=(V // _RB,),
        in_specs=[
            pl.BlockSpec((_RB, C), lambda i: (i, 0)),
            pl.BlockSpec((_RB, C), lambda i: (i, 0)),
            pl.BlockSpec((_RB, C), lambda i: (i, 0)),
            pl.BlockSpec((_RB, C), lambda i: (i + V // _RB, 0)),
            pl.BlockSpec((3, C, FOUT), lambda i: (0, 0, 0)),
        ],
        out_specs=pl.BlockSpec((_RB, FOUT), lambda i: (i, 0)),
        out_shape=jax.ShapeDtypeStruct((V, FOUT), jnp.float32),
    )(x0, x1, p2, p2, weight)


def _pad_edges(a, fill):
    per = a.reshape(NW, EPT)
    pad = jnp.full((NW, NPC * CHUNK - EPT), fill, a.dtype)
    return jnp.concatenate([per, pad], axis=1).reshape(NW, NPC, CHUNK)


def kernel(inputs, edge_index, edge_weight, weight):
    B, Fin, V_, X, Y, Z = inputs.shape
    K, _, Fout = weight.shape
    x0 = inputs.reshape(Fin, V_).T
    row = _pad_edges(edge_index[0], 0)
    col = _pad_edges(edge_index[1], 0)
    w3 = _pad_edges(edge_weight, 0.0)
    p1 = _spmv_sc(x0, row, col, w3)
    x1 = _combine(p1)
    p2 = _spmv_sc(x1, row, col, w3)
    y = _mix(x0, x1, p2, weight)
    return y.T.reshape(B, Fout, V_, X, Y, Z)
